# baseline (speedup 1.0000x reference)
import dataclasses
import functools

import jax
import jax.numpy as jnp
from jax import lax
from jax.experimental import pallas as pl
from jax.experimental.pallas import tpu as pltpu
from jax.experimental.pallas import tpu_sc as plsc

_USER = 25000
_ITEM = 25000
_N = 50000
_EMB = 32
_LAYERS = 3
_ADJ_NNZ = 1600000
_S_NNZ = 400000

_NC = 2
_NS = 16
_NW = _NC * _NS
_CH = 256


def _make_spmm(ncw, n_out_pad):
  nb = ncw // 4
  assert ncw % 4 == 0 and nb % 2 == 0
  rp = n_out_pad // _NS
  nzf, nzr = divmod(rp, _CH)

  mesh = plsc.VectorSubcoreMesh(core_axis_name="c", subcore_axis_name="s")
  cp = pltpu.CompilerParams()
  fields = pltpu.CompilerParams.__dataclass_fields__
  if "needs_layout_passes" in fields:
    cp = dataclasses.replace(cp, needs_layout_passes=False)
  if "use_tc_tiling_on_sc" in fields:
    cp = dataclasses.replace(cp, use_tc_tiling_on_sc=False)

  @functools.partial(
      pl.kernel,
      out_type=jax.ShapeDtypeStruct((_NC, n_out_pad, _EMB), jnp.float32),
      mesh=mesh,
      compiler_params=cp,
      scratch_types=[
          pltpu.VMEM_SHARED((n_out_pad, _EMB), jnp.float32),
          pltpu.VMEM((4, _CH), jnp.int32),
          pltpu.VMEM((4, _CH), jnp.int32),
          pltpu.VMEM((4, _CH), jnp.int32),
          pltpu.VMEM((4, _CH), jnp.int32),
          pltpu.VMEM((4, _CH), jnp.float32),
          pltpu.VMEM((4, _CH), jnp.float32),
          pltpu.VMEM((_CH, _EMB), jnp.float32),
          pltpu.VMEM((_CH, _EMB), jnp.float32),
          pltpu.VMEM((_CH,), jnp.int32),
          pltpu.SemaphoreType.DMA,
          pltpu.SemaphoreType.DMA,
          pltpu.SemaphoreType.DMA,
          pltpu.SemaphoreType.DMA,
          pltpu.SemaphoreType.DMA,
          pltpu.SemaphoreType.DMA,
      ],
  )
  def spmm(x_hbm, dst_hbm, src_hbm, vals_hbm, zeros_hbm, part_hbm,
           acc_sh, dsti0, dsti1, srci0, srci1, vals0, vals1,
           rows0, rows1, dumidx, isem0, isem1,
           gsem0, gsem1, ssem0, ssem1):
    cid = lax.axis_index("c")
    sid = lax.axis_index("s")
    w = sid * _NC + cid
    dsti = (dsti0, dsti1)
    srci = (srci0, srci1)
    vals = (vals0, vals1)
    rows = (rows0, rows1)
    isem = (isem0, isem1)
    gsem = (gsem0, gsem1)
    ssem = (ssem0, ssem1)
    cbase = w * ncw

    zbase = sid * rp

    @pl.loop(0, nzf)
    def _(j):
      pltpu.sync_copy(zeros_hbm, acc_sh.at[pl.ds(zbase + j * _CH, _CH)])

    if nzr:
      pltpu.sync_copy(zeros_hbm.at[pl.ds(0, nzr)],
                      acc_sh.at[pl.ds(zbase + nzf * _CH, nzr)])

    plsc.subcore_barrier()

    lane = lax.broadcasted_iota(jnp.int32, (16,), 0)
    lane16 = lane + 16

    def issue_idx(b, p):
      blk = pl.ds(cbase + b * 4, 4)
      pltpu.async_copy(src_hbm.at[blk], srci[p], isem[p])
      pltpu.async_copy(vals_hbm.at[blk], vals[p], isem[p])
      pltpu.async_copy(dst_hbm.at[blk], dsti[p], isem[p])

    def wait_idx(p):
      pltpu.make_async_copy(dst_hbm.at[pl.ds(0, 4)], dsti[p], isem[p]).wait()
      pltpu.make_async_copy(src_hbm.at[pl.ds(0, 4)], srci[p], isem[p]).wait()
      pltpu.make_async_copy(vals_hbm.at[pl.ds(0, 4)], vals[p], isem[p]).wait()

    def issue_gather(p, jp, j):
      pltpu.async_copy(x_hbm.at[srci[p].at[j]], rows[jp], gsem[jp])

    def wait_gather(jp):
      pltpu.make_async_copy(x_hbm.at[dumidx], rows[jp], gsem[jp]).wait()

    def wait_scatter(jp):
      pltpu.make_async_copy(rows[jp], acc_sh.at[dumidx], ssem[jp]).wait()

    def scale(rbuf, vref, j):
      @pl.loop(0, _CH // 16)
      def _(g):
        b = g * 16
        jr = jnp.zeros((16,), jnp.int32) + j
        for e in range(16):
          r = jnp.zeros((16,), jnp.int32) + (b + e)
          sv = plsc.load_gather(vref, [jr, r])
          h0 = plsc.load_gather(rbuf, [r, lane])
          h1 = plsc.load_gather(rbuf, [r, lane16])
          plsc.store_scatter(rbuf, [r, lane], h0 * sv)
          plsc.store_scatter(rbuf, [r, lane16], h1 * sv)

    z16 = jnp.zeros((16,), jnp.int32)
    @pl.loop(0, _CH // 16)
    def _(g):
      dumidx[pl.ds(g * 16, 16)] = z16

    issue_idx(0, 0)
    pltpu.async_copy(zeros_hbm, rows1, ssem1)
    wait_idx(0)
    issue_gather(0, 0, 0)

    @pl.loop(0, nb, step=2)
    def _(bi):
      for half in range(2):
        b = bi + half
        p = half
        for j in range(4):
          jp = j & 1
          wait_gather(jp)
          wait_scatter(1 - jp)
          if j < 3:
            issue_gather(p, 1 - jp, j + 1)
          else:
            @pl.when(b < nb - 1)
            def _():
              wait_idx(1 - p)
              issue_gather(1 - p, 1 - jp, 0)
          if j == 1:
            @pl.when(b + 1 < nb)
            def _():
              issue_idx(b + 1, 1 - p)
          scale(rows[jp], vals[p], j)
          pltpu.async_copy(rows[jp], acc_sh.at[dsti[p].at[j]], ssem[jp],
                           add=True)

    wait_scatter(1)

    plsc.subcore_barrier()
    _dump(acc_sh, part_hbm, cid, zbase)

  def _dump(acc_sh, part_hbm, cid, zbase):
    @pl.loop(0, nzf)
    def _(j):
      o = zbase + j * _CH
      pltpu.sync_copy(acc_sh.at[pl.ds(o, _CH)],
                      part_hbm.at[cid].at[pl.ds(o, _CH)])

    if nzr:
      o = zbase + nzf * _CH
      pltpu.sync_copy(acc_sh.at[pl.ds(o, nzr)],
                      part_hbm.at[cid].at[pl.ds(o, nzr)])

  return spmm


_N_PAD = 50048
_S_PAD = 25088
_ADJ_NCW = 208
_S_NCW = 64
_spmm_adj = _make_spmm(_ADJ_NCW, _N_PAD)
_spmm_s = _make_spmm(_S_NCW, _S_PAD)


def _prep_edges(indices, values, ncw):
  total = ncw * _NW * _CH
  pad = total - values.shape[0]
  dst = jnp.pad(indices[0], (0, pad)).reshape(-1, _CH)
  src = jnp.pad(indices[1], (0, pad)).reshape(-1, _CH)
  val = jnp.pad(values, (0, pad)).reshape(-1, _CH)
  return dst, src, val

_TB = 1000


def _tc_update_users(cur, p):
  nu = _USER // _TB

  def body(cur_ref, p_ref, o_ref):
    i = pl.program_id(0)

    @pl.when(i < nu)
    def _():
      o_ref[...] = cur_ref[...] + p_ref[0] + p_ref[1]

    @pl.when(i >= nu)
    def _():
      o_ref[...] = cur_ref[...]

  return pl.pallas_call(
      body,
      grid=(_N // _TB,),
      in_specs=[
          pl.BlockSpec((_TB, _EMB), lambda i: (i, 0)),
          pl.BlockSpec((2, _TB, _EMB), lambda i: (0, jnp.minimum(i, nu - 1), 0)),
      ],
      out_specs=pl.BlockSpec((_TB, _EMB), lambda i: (i, 0)),
      out_shape=jax.ShapeDtypeStruct((_N, _EMB), jnp.float32),
  )(cur, p)


def _tc_combine(p, acc, final):
  def body(p_ref, acc_ref, cur_ref, acco_ref):
    s = p_ref[0] + p_ref[1]
    cur_ref[...] = s
    a = acc_ref[...] + s
    if final:
      a = a * 0.25
    acco_ref[...] = a

  return pl.pallas_call(
      body,
      grid=(_N // _TB,),
      in_specs=[
          pl.BlockSpec((2, _TB, _EMB), lambda i: (0, i, 0)),
          pl.BlockSpec((_TB, _EMB), lambda i: (i, 0)),
      ],
      out_specs=[
          pl.BlockSpec((_TB, _EMB), lambda i: (i, 0)),
          pl.BlockSpec((_TB, _EMB), lambda i: (i, 0)),
      ],
      out_shape=[jax.ShapeDtypeStruct((_N, _EMB), jnp.float32)] * 2,
  )(p, acc)


def kernel(user_emb, item_emb, adj_indices, adj_values, s_indices, s_values):
  ego0 = jnp.concatenate([user_emb, item_emb], axis=0)
  zeros = jnp.zeros((_CH, _EMB), jnp.float32)
  adj_dst, adj_src, adj_val = _prep_edges(adj_indices, adj_values, _ADJ_NCW)
  s_dst, s_src, s_val = _prep_edges(s_indices, s_values, _S_NCW)

  cur = ego0
  acc = ego0
  for k in range(_LAYERS):
    sp = _spmm_s(cur, s_dst, s_src, s_val, zeros)
    cur = _tc_update_users(cur, sp)
    ap = _spmm_adj(cur, adj_dst, adj_src, adj_val, zeros)
    cur, acc = _tc_combine(ap, acc, final=(k == _LAYERS - 1))
  return acc[:_USER], acc[_USER:]


# speedup vs baseline: 1.6580x; 1.6580x over previous
import dataclasses
import functools

import jax
import jax.numpy as jnp
from jax import lax
from jax.experimental import pallas as pl
from jax.experimental.pallas import tpu as pltpu
from jax.experimental.pallas import tpu_sc as plsc

_USER = 25000
_ITEM = 25000
_N = 50000
_EMB = 32
_LAYERS = 3

_NC = 2
_NS = 16
_NW = _NC * _NS
_CH = 128


def _make_spmm(ncw, n_out_pad):
  nb = ncw // 8
  assert ncw % 8 == 0 and nb % 2 == 0
  rp = n_out_pad // _NS
  nzf, nzr = divmod(rp, _CH)

  mesh = plsc.VectorSubcoreMesh(core_axis_name="c", subcore_axis_name="s")
  cp = pltpu.CompilerParams()
  fields = pltpu.CompilerParams.__dataclass_fields__
  if "needs_layout_passes" in fields:
    cp = dataclasses.replace(cp, needs_layout_passes=False)
  if "use_tc_tiling_on_sc" in fields:
    cp = dataclasses.replace(cp, use_tc_tiling_on_sc=False)

  @functools.partial(
      pl.kernel,
      out_type=jax.ShapeDtypeStruct((_NC, n_out_pad, _EMB), jnp.float32),
      mesh=mesh,
      compiler_params=cp,
      scratch_types=[
          pltpu.VMEM_SHARED((n_out_pad, _EMB), jnp.float32),
          pltpu.VMEM((8, _CH), jnp.int32),
          pltpu.VMEM((8, _CH), jnp.int32),
          pltpu.VMEM((8, _CH), jnp.int32),
          pltpu.VMEM((8, _CH), jnp.int32),
          pltpu.VMEM((8, _CH), jnp.float32),
          pltpu.VMEM((8, _CH), jnp.float32),
          pltpu.VMEM((_CH, _EMB), jnp.float32),
          pltpu.VMEM((_CH, _EMB), jnp.float32),
          pltpu.VMEM((_CH, _EMB), jnp.float32),
          pltpu.VMEM((_CH, _EMB), jnp.float32),
          pltpu.VMEM((_CH,), jnp.int32),
          pltpu.SemaphoreType.DMA,
          pltpu.SemaphoreType.DMA,
          pltpu.SemaphoreType.DMA,
          pltpu.SemaphoreType.DMA,
          pltpu.SemaphoreType.DMA,
          pltpu.SemaphoreType.DMA,
          pltpu.SemaphoreType.DMA,
          pltpu.SemaphoreType.DMA,
          pltpu.SemaphoreType.DMA,
          pltpu.SemaphoreType.DMA,
      ],
  )
  def spmm(x_hbm, dst_hbm, src_hbm, vals_hbm, zeros_hbm, part_hbm,
           acc_sh, dsti0, dsti1, srci0, srci1, vals0, vals1,
           rows0, rows1, rows2, rows3, dumidx, isem0, isem1,
           gsem0, gsem1, gsem2, gsem3, ssem0, ssem1, ssem2, ssem3):
    cid = lax.axis_index("c")
    sid = lax.axis_index("s")
    w = sid * _NC + cid
    dsti = (dsti0, dsti1)
    srci = (srci0, srci1)
    vals = (vals0, vals1)
    rows = (rows0, rows1, rows2, rows3)
    isem = (isem0, isem1)
    gsem = (gsem0, gsem1, gsem2, gsem3)
    ssem = (ssem0, ssem1, ssem2, ssem3)
    cbase = w * ncw

    zbase = sid * rp

    @pl.loop(0, nzf)
    def _(j):
      pltpu.sync_copy(zeros_hbm, acc_sh.at[pl.ds(zbase + j * _CH, _CH)])

    if nzr:
      pltpu.sync_copy(zeros_hbm.at[pl.ds(0, nzr)],
                      acc_sh.at[pl.ds(zbase + nzf * _CH, nzr)])

    plsc.subcore_barrier()

    lane = lax.broadcasted_iota(jnp.int32, (16,), 0)
    lane16 = lane + 16

    def issue_idx_sv(b, p):
      blk = pl.ds(cbase + b * 8, 8)
      pltpu.async_copy(src_hbm.at[blk], srci[p], isem[p])
      pltpu.async_copy(vals_hbm.at[blk], vals[p], isem[p])

    def issue_idx_d(b, p):
      blk = pl.ds(cbase + b * 8, 8)
      pltpu.async_copy(dst_hbm.at[blk], dsti[p], isem[p])

    def wait_idx(p):
      pltpu.make_async_copy(dst_hbm.at[pl.ds(0, 8)], dsti[p], isem[p]).wait()
      pltpu.make_async_copy(src_hbm.at[pl.ds(0, 8)], srci[p], isem[p]).wait()
      pltpu.make_async_copy(vals_hbm.at[pl.ds(0, 8)], vals[p], isem[p]).wait()

    def issue_gather(p, jp, j):
      pltpu.async_copy(x_hbm.at[srci[p].at[j]], rows[jp], gsem[jp])

    def wait_gather(jp):
      pltpu.make_async_copy(x_hbm.at[dumidx], rows[jp], gsem[jp]).wait()

    def wait_scatter(jp):
      pltpu.make_async_copy(rows[jp], acc_sh.at[dumidx], ssem[jp]).wait()

    def scale(rbuf, vref, j):
      @pl.loop(0, _CH // 16)
      def _(g):
        b = g * 16
        jr = jnp.zeros((16,), jnp.int32) + j
        for e in range(16):
          r = jnp.zeros((16,), jnp.int32) + (b + e)
          sv = plsc.load_gather(vref, [jr, r])
          h0 = plsc.load_gather(rbuf, [r, lane])
          h1 = plsc.load_gather(rbuf, [r, lane16])
          plsc.store_scatter(rbuf, [r, lane], h0 * sv)
          plsc.store_scatter(rbuf, [r, lane16], h1 * sv)

    z16 = jnp.zeros((16,), jnp.int32)

    @pl.loop(0, _CH // 16)
    def _(g):
      dumidx[pl.ds(g * 16, 16)] = z16

    issue_idx_sv(0, 0)
    issue_idx_d(0, 0)
    pltpu.async_copy(zeros_hbm, rows3, ssem3)
    wait_idx(0)
    issue_gather(0, 0, 0)
    issue_gather(0, 1, 1)
    issue_gather(0, 2, 2)

    @pl.loop(0, nb, step=2)
    def _(bi):
      for half in range(2):
        b = bi + half
        p = half
        for j in range(8):
          q = j & 3
          q2 = (j + 3) & 3
          wait_gather(q)
          scale(rows[q], vals[p], j)
          pltpu.async_copy(rows[q], acc_sh.at[dsti[p].at[j]], ssem[q],
                           add=True)
          if j == 2:
            @pl.when(b + 1 < nb)
            def _():
              issue_idx_sv(b + 1, 1 - p)
          if j < 5:
            wait_scatter(q2)
            issue_gather(p, q2, j + 3)
            if j == 4:
              @pl.when(b + 1 < nb)
              def _():
                issue_idx_d(b + 1, 1 - p)
          else:
            @pl.when(b < nb - 1)
            def _():
              if j == 5:
                wait_idx(1 - p)
              wait_scatter(q2)
              issue_gather(1 - p, q2, j - 5)

    wait_scatter(0)
    wait_scatter(1)
    wait_scatter(2)
    wait_scatter(3)

    plsc.subcore_barrier()

    @pl.loop(0, nzf)
    def _(j):
      o = zbase + j * _CH
      pltpu.sync_copy(acc_sh.at[pl.ds(o, _CH)],
                      part_hbm.at[cid].at[pl.ds(o, _CH)])

    if nzr:
      o = zbase + nzf * _CH
      pltpu.sync_copy(acc_sh.at[pl.ds(o, nzr)],
                      part_hbm.at[cid].at[pl.ds(o, nzr)])

  return spmm


_N_PAD = 50048
_S_PAD = 25088
_ADJ_NCW = 400
_S_NCW = 112
_spmm_adj = _make_spmm(_ADJ_NCW, _N_PAD)
_spmm_s = _make_spmm(_S_NCW, _S_PAD)


def _prep_edges(indices, values, ncw):
  total = ncw * _NW * _CH
  pad = total - values.shape[0]
  dst = jnp.pad(indices[0], (0, pad)).reshape(-1, _CH)
  src = jnp.pad(indices[1], (0, pad)).reshape(-1, _CH)
  val = jnp.pad(values, (0, pad)).reshape(-1, _CH)
  return dst, src, val


_TB = 1000


def _tc_update_users(cur, p):
  nu = _USER // _TB

  def body(cur_ref, p_ref, o_ref):
    i = pl.program_id(0)

    @pl.when(i < nu)
    def _():
      o_ref[...] = cur_ref[...] + p_ref[0] + p_ref[1]

    @pl.when(i >= nu)
    def _():
      o_ref[...] = cur_ref[...]

  return pl.pallas_call(
      body,
      grid=(_N // _TB,),
      in_specs=[
          pl.BlockSpec((_TB, _EMB), lambda i: (i, 0)),
          pl.BlockSpec((2, _TB, _EMB), lambda i: (0, jnp.minimum(i, nu - 1), 0)),
      ],
      out_specs=pl.BlockSpec((_TB, _EMB), lambda i: (i, 0)),
      out_shape=jax.ShapeDtypeStruct((_N, _EMB), jnp.float32),
  )(cur, p)


def _tc_combine(p, acc, final):
  def body(p_ref, acc_ref, cur_ref, acco_ref):
    s = p_ref[0] + p_ref[1]
    cur_ref[...] = s
    a = acc_ref[...] + s
    if final:
      a = a * 0.25
    acco_ref[...] = a

  return pl.pallas_call(
      body,
      grid=(_N // _TB,),
      in_specs=[
          pl.BlockSpec((2, _TB, _EMB), lambda i: (0, i, 0)),
          pl.BlockSpec((_TB, _EMB), lambda i: (i, 0)),
      ],
      out_specs=[
          pl.BlockSpec((_TB, _EMB), lambda i: (i, 0)),
          pl.BlockSpec((_TB, _EMB), lambda i: (i, 0)),
      ],
      out_shape=[jax.ShapeDtypeStruct((_N, _EMB), jnp.float32)] * 2,
  )(p, acc)


def kernel(user_emb, item_emb, adj_indices, adj_values, s_indices, s_values):
  ego0 = jnp.concatenate([user_emb, item_emb], axis=0)
  zeros = jnp.zeros((_CH, _EMB), jnp.float32)
  adj_dst, adj_src, adj_val = _prep_edges(adj_indices, adj_values, _ADJ_NCW)
  s_dst, s_src, s_val = _prep_edges(s_indices, s_values, _S_NCW)

  cur = ego0
  acc = ego0
  for k in range(_LAYERS):
    sp = _spmm_s(cur, s_dst, s_src, s_val, zeros)
    cur = _tc_update_users(cur, sp)
    ap = _spmm_adj(cur, adj_dst, adj_src, adj_val, zeros)
    cur, acc = _tc_combine(ap, acc, final=(k == _LAYERS - 1))
  return acc[:_USER], acc[_USER:]


# speedup vs baseline: 2.0818x; 1.2557x over previous
import dataclasses
import functools

import jax
import jax.numpy as jnp
from jax import lax
from jax.experimental import pallas as pl
from jax.experimental.pallas import tpu as pltpu
from jax.experimental.pallas import tpu_sc as plsc

_USER = 25000
_ITEM = 25000
_N = 50000
_EMB = 32
_LAYERS = 3

_NC = 2
_NS = 16
_NW = _NC * _NS
_CH = 128


def _make_spmm(ncw, n_out_pad):
  nb = ncw // 8
  assert ncw % 8 == 0 and nb % 2 == 0
  rp = n_out_pad // _NS
  nzf, nzr = divmod(rp, _CH)

  mesh = plsc.VectorSubcoreMesh(core_axis_name="c", subcore_axis_name="s")
  cp = pltpu.CompilerParams()
  fields = pltpu.CompilerParams.__dataclass_fields__
  if "needs_layout_passes" in fields:
    cp = dataclasses.replace(cp, needs_layout_passes=False)
  if "use_tc_tiling_on_sc" in fields:
    cp = dataclasses.replace(cp, use_tc_tiling_on_sc=False)

  @functools.partial(
      pl.kernel,
      out_type=jax.ShapeDtypeStruct((_NC, n_out_pad, _EMB), jnp.float32),
      mesh=mesh,
      compiler_params=cp,
      scratch_types=[
          pltpu.VMEM_SHARED((n_out_pad, _EMB), jnp.float32),
          pltpu.VMEM((8, _CH), jnp.int32),
          pltpu.VMEM((8, _CH), jnp.int32),
          pltpu.VMEM((8, _CH), jnp.int32),
          pltpu.VMEM((8, _CH), jnp.int32),
          pltpu.VMEM((8, _CH), jnp.int32),
          pltpu.VMEM((8, _CH), jnp.int32),
          pltpu.VMEM((_CH, _EMB // 2), jnp.int32),
          pltpu.VMEM((_CH, _EMB // 2), jnp.int32),
          pltpu.VMEM((_CH, _EMB // 2), jnp.int32),
          pltpu.VMEM((_CH, _EMB // 2), jnp.int32),
          pltpu.VMEM((_CH, _EMB), jnp.float32),
          pltpu.VMEM((_CH, _EMB), jnp.float32),
          pltpu.VMEM((_CH,), jnp.int32),
          pltpu.SemaphoreType.DMA,
          pltpu.SemaphoreType.DMA,
          pltpu.SemaphoreType.DMA,
          pltpu.SemaphoreType.DMA,
          pltpu.SemaphoreType.DMA,
          pltpu.SemaphoreType.DMA,
          pltpu.SemaphoreType.DMA,
          pltpu.SemaphoreType.DMA,
      ],
  )
  def spmm(x_hbm, dst_hbm, src_hbm, vals_hbm, zeros_hbm, part_hbm,
           acc_sh, dsti0, dsti1, srci0, srci1, vals0, vals1,
           rows0, rows1, rows2, rows3, frows0, frows1, dumidx, isem0, isem1,
           gsem0, gsem1, gsem2, gsem3, ssem0, ssem1):
    cid = lax.axis_index("c")
    sid = lax.axis_index("s")
    w = sid * _NC + cid
    dsti = (dsti0, dsti1)
    srci = (srci0, srci1)
    vals = (vals0, vals1)
    rows = (rows0, rows1, rows2, rows3)
    frows = (frows0, frows1)
    isem = (isem0, isem1)
    gsem = (gsem0, gsem1, gsem2, gsem3)
    ssem = (ssem0, ssem1)
    cbase = w * ncw

    zbase = sid * rp

    @pl.loop(0, nzf)
    def _(j):
      pltpu.sync_copy(zeros_hbm, acc_sh.at[pl.ds(zbase + j * _CH, _CH)])

    if nzr:
      pltpu.sync_copy(zeros_hbm.at[pl.ds(0, nzr)],
                      acc_sh.at[pl.ds(zbase + nzf * _CH, nzr)])

    plsc.subcore_barrier()

    lane = lax.broadcasted_iota(jnp.int32, (16,), 0)
    lane16 = lane + 16

    def issue_idx_sv(b, p):
      blk = pl.ds(cbase + b * 8, 8)
      pltpu.async_copy(src_hbm.at[blk], srci[p], isem[p])
      pltpu.async_copy(vals_hbm.at[blk], vals[p], isem[p])

    def issue_idx_d(b, p):
      blk = pl.ds(cbase + b * 8, 8)
      pltpu.async_copy(dst_hbm.at[blk], dsti[p], isem[p])

    def wait_idx(p):
      pltpu.make_async_copy(dst_hbm.at[pl.ds(0, 8)], dsti[p], isem[p]).wait()
      pltpu.make_async_copy(src_hbm.at[pl.ds(0, 8)], srci[p], isem[p]).wait()
      pltpu.make_async_copy(vals_hbm.at[pl.ds(0, 8)], vals[p], isem[p]).wait()

    def issue_gather(p, jp, j):
      pltpu.async_copy(x_hbm.at[srci[p].at[j]], rows[jp], gsem[jp])

    def wait_gather(jp):
      pltpu.make_async_copy(x_hbm.at[dumidx], rows[jp], gsem[jp]).wait()

    def wait_scatter(sp):
      pltpu.make_async_copy(frows[sp], acc_sh.at[dumidx], ssem[sp]).wait()

    lane2e = lane * 2
    lane2o = lane2e + 1

    def scale(rbuf, vref, j, fbuf):
      @pl.loop(0, _CH // 16)
      def _(g):
        b = g * 16
        jr = jnp.zeros((16,), jnp.int32) + j
        for e in range(16):
          r = jnp.zeros((16,), jnp.int32) + (b + e)
          svi = plsc.load_gather(vref, [jr, r])
          sv = plsc.bitcast(svi, jnp.bfloat16)
          hi = plsc.load_gather(rbuf, [r, lane])
          h = plsc.bitcast(hi, jnp.bfloat16)
          prod = h * sv
          fa, fb = plsc.unpack(prod, format=plsc.PackFormat.INTERLEAVED)
          plsc.store_scatter(fbuf, [r, lane2e], fa)
          plsc.store_scatter(fbuf, [r, lane2o], fb)

    z16 = jnp.zeros((16,), jnp.int32)

    @pl.loop(0, _CH // 16)
    def _(g):
      dumidx[pl.ds(g * 16, 16)] = z16

    issue_idx_sv(0, 0)
    issue_idx_d(0, 0)
    pltpu.async_copy(zeros_hbm, frows0, ssem0)
    pltpu.async_copy(zeros_hbm, frows1, ssem1)
    wait_idx(0)
    issue_gather(0, 0, 0)
    issue_gather(0, 1, 1)
    issue_gather(0, 2, 2)

    @pl.loop(0, nb, step=2)
    def _(bi):
      for half in range(2):
        b = bi + half
        p = half
        for j in range(8):
          q = j & 3
          q2 = (j + 3) & 3
          s2 = j & 1
          wait_gather(q)
          if j < 5:
            issue_gather(p, q2, j + 3)
          else:
            @pl.when(b < nb - 1)
            def _():
              if j == 5:
                wait_idx(1 - p)
              issue_gather(1 - p, q2, j - 5)
          wait_scatter(s2)
          scale(rows[q], vals[p], j, frows[s2])
          pltpu.async_copy(frows[s2], acc_sh.at[dsti[p].at[j]], ssem[s2],
                           add=True)
          if j == 2:
            @pl.when(b + 1 < nb)
            def _():
              issue_idx_sv(b + 1, 1 - p)
          if j == 4:
            @pl.when(b + 1 < nb)
            def _():
              issue_idx_d(b + 1, 1 - p)

    wait_scatter(0)
    wait_scatter(1)

    plsc.subcore_barrier()

    @pl.loop(0, nzf)
    def _(j):
      o = zbase + j * _CH
      pltpu.sync_copy(acc_sh.at[pl.ds(o, _CH)],
                      part_hbm.at[cid].at[pl.ds(o, _CH)])

    if nzr:
      o = zbase + nzf * _CH
      pltpu.sync_copy(acc_sh.at[pl.ds(o, nzr)],
                      part_hbm.at[cid].at[pl.ds(o, nzr)])

  return spmm


_N_PAD = 50048
_S_PAD = 25088
_ADJ_NCW = 400
_S_NCW = 112
_spmm_adj = _make_spmm(_ADJ_NCW, _N_PAD)
_spmm_s = _make_spmm(_S_NCW, _S_PAD)


def _prep_edges(indices, values, ncw):
  total = ncw * _NW * _CH
  pad = total - values.shape[0]
  dst = jnp.pad(indices[0], (0, pad)).reshape(-1, _CH)
  src = jnp.pad(indices[1], (0, pad)).reshape(-1, _CH)
  vb = jnp.pad(values, (0, pad)).astype(jnp.bfloat16)
  val = jax.lax.bitcast_convert_type(
      jnp.stack([vb, vb], axis=-1), jnp.int32).reshape(-1, _CH)
  return dst, src, val


_TB = 1000


def _tc_update_users(cur, p):
  nu = _USER // _TB

  def body(cur_ref, p_ref, o_ref):
    i = pl.program_id(0)

    @pl.when(i < nu)
    def _():
      o_ref[...] = cur_ref[...] + p_ref[0] + p_ref[1]

    @pl.when(i >= nu)
    def _():
      o_ref[...] = cur_ref[...]

  return pl.pallas_call(
      body,
      grid=(_N // _TB,),
      in_specs=[
          pl.BlockSpec((_TB, _EMB), lambda i: (i, 0)),
          pl.BlockSpec((2, _TB, _EMB), lambda i: (0, jnp.minimum(i, nu - 1), 0)),
      ],
      out_specs=pl.BlockSpec((_TB, _EMB), lambda i: (i, 0)),
      out_shape=jax.ShapeDtypeStruct((_N, _EMB), jnp.float32),
  )(cur, p)


def _tc_combine(p, acc, final):
  def body(p_ref, acc_ref, cur_ref, acco_ref):
    s = p_ref[0] + p_ref[1]
    cur_ref[...] = s
    a = acc_ref[...] + s
    if final:
      a = a * 0.25
    acco_ref[...] = a

  return pl.pallas_call(
      body,
      grid=(_N // _TB,),
      in_specs=[
          pl.BlockSpec((2, _TB, _EMB), lambda i: (0, i, 0)),
          pl.BlockSpec((_TB, _EMB), lambda i: (i, 0)),
      ],
      out_specs=[
          pl.BlockSpec((_TB, _EMB), lambda i: (i, 0)),
          pl.BlockSpec((_TB, _EMB), lambda i: (i, 0)),
      ],
      out_shape=[jax.ShapeDtypeStruct((_N, _EMB), jnp.float32)] * 2,
  )(p, acc)


def kernel(user_emb, item_emb, adj_indices, adj_values, s_indices, s_values):
  ego0 = jnp.concatenate([user_emb, item_emb], axis=0)
  zeros = jnp.zeros((_CH, _EMB), jnp.float32)
  adj_dst, adj_src, adj_val = _prep_edges(adj_indices, adj_values, _ADJ_NCW)
  s_dst, s_src, s_val = _prep_edges(s_indices, s_values, _S_NCW)

  def bitview(x):
    xb = x.astype(jnp.bfloat16)
    return jax.lax.bitcast_convert_type(xb.reshape(_N, _EMB // 2, 2),
                                        jnp.int32)

  cur = ego0
  acc = ego0
  for k in range(_LAYERS):
    sp = _spmm_s(bitview(cur), s_dst, s_src, s_val, zeros)
    cur = _tc_update_users(cur, sp)
    ap = _spmm_adj(bitview(cur), adj_dst, adj_src, adj_val, zeros)
    cur, acc = _tc_combine(ap, acc, final=(k == _LAYERS - 1))
  return acc[:_USER], acc[_USER:]
